# baseline (speedup 1.0000x reference)
import functools

import jax
import jax.numpy as jnp
from jax import lax
from jax.experimental import pallas as pl
from jax.experimental.pallas import tpu as pltpu
from jax.experimental.pallas import tpu_sc as plsc

NC = 2
NS = 16
NW = NC * NS
L = 16

BATCH = 16384
B_PER_W = BATCH // NW
ROWS_PER_W = B_PER_W // 128


def _gather_body(uv, iv, uidx_hbm, iidx_hbm, uflat_hbm, iflat_hbm,
                 uout_hbm, iout_hbm, uidx_v, iidx_v, uval_v, ival_v, sem):
    wid = lax.axis_index("s") * NC + lax.axis_index("c")
    row0 = wid * ROWS_PER_W

    pltpu.sync_copy(uidx_hbm.at[pl.ds(row0, ROWS_PER_W), :], uidx_v)
    pltpu.sync_copy(iidx_hbm.at[pl.ds(row0, ROWS_PER_W), :], iidx_v)

    for j in range(ROWS_PER_W):
        for k in range(128 // L):
            u = uidx_v[j, pl.ds(k * L, L)]
            uidx_v[j, pl.ds(k * L, L)] = jnp.clip(u, 0, uv - 1) * 64
            i = iidx_v[j, pl.ds(k * L, L)]
            iidx_v[j, pl.ds(k * L, L)] = jnp.clip(i, 0, iv - 1) * 64

    copies = []
    for j in range(ROWS_PER_W):
        copies.append(pltpu.make_async_copy(
            uflat_hbm.at[uidx_v.at[j]], uval_v.at[j], sem))
        copies.append(pltpu.make_async_copy(
            iflat_hbm.at[iidx_v.at[j]], ival_v.at[j], sem))
    for c in copies:
        c.start()
    for c in copies:
        c.wait()

    pltpu.sync_copy(uval_v, uout_hbm.at[pl.ds(row0, ROWS_PER_W), :])
    pltpu.sync_copy(ival_v, iout_hbm.at[pl.ds(row0, ROWS_PER_W), :])


def _lookup_sc(uidx, iidx, uflat, iflat, uv, iv):
    mesh = plsc.VectorSubcoreMesh(core_axis_name="c", subcore_axis_name="s")
    f = pl.kernel(
        functools.partial(_gather_body, uv, iv),
        mesh=mesh,
        out_type=(
            jax.ShapeDtypeStruct((BATCH // 128, 128), jnp.float32),
            jax.ShapeDtypeStruct((BATCH // 128, 128), jnp.float32),
        ),
        scratch_types=[
            pltpu.VMEM((ROWS_PER_W, 128), jnp.int32),
            pltpu.VMEM((ROWS_PER_W, 128), jnp.int32),
            pltpu.VMEM((ROWS_PER_W, 128), jnp.float32),
            pltpu.VMEM((ROWS_PER_W, 128), jnp.float32),
            pltpu.SemaphoreType.DMA,
        ],
    )
    return f(uidx, iidx, uflat, iflat)


def kernel(x, uid_weight, iid_weight):
    uidx = x[:, 0].reshape(BATCH // 128, 128)
    iidx = x[:, 1].reshape(BATCH // 128, 128)
    uflat = uid_weight.reshape(-1)
    iflat = iid_weight.reshape(-1)
    uvals, ivals = _lookup_sc(uidx, iidx, uflat, iflat,
                              uid_weight.shape[0], iid_weight.shape[0])
    return jnp.stack([uvals.reshape(BATCH), ivals.reshape(BATCH)], axis=1)


# speedup vs baseline: 1.1764x; 1.1764x over previous
import functools

import jax
import jax.numpy as jnp
from jax import lax
from jax.experimental import pallas as pl
from jax.experimental.pallas import tpu as pltpu
from jax.experimental.pallas import tpu_sc as plsc

NC = 2
NS = 16
NW = NC * NS
L = 16

BATCH = 16384
B_PER_W = BATCH // NW
W = 64
NWIN = B_PER_W // W


def _gather_body(uv, iv, uidx_hbm, iidx_hbm, utab_hbm, itab_hbm,
                 uout_hbm, iout_hbm, uidx_v, iidx_v, ubuf_v, ibuf_v,
                 usem, isem, osem):
    wid = lax.axis_index("s") * NC + lax.axis_index("c")
    row0 = wid * NWIN
    base = wid * B_PER_W

    pltpu.sync_copy(uidx_hbm.at[pl.ds(row0, NWIN), :], uidx_v)
    pltpu.sync_copy(iidx_hbm.at[pl.ds(row0, NWIN), :], iidx_v)

    for j in range(NWIN):
        for k in range(W // L):
            u = uidx_v[j, pl.ds(k * L, L)]
            uidx_v[j, pl.ds(k * L, L)] = jnp.clip(u, 0, uv - 1)
            i = iidx_v[j, pl.ds(k * L, L)]
            iidx_v[j, pl.ds(k * L, L)] = jnp.clip(i, 0, iv - 1)

    def window(w, carry):
        pos = pl.multiple_of(base + w * W, W)
        ucopies = []
        for k in range(W // L):
            vu = uidx_v[w, pl.ds(k * L, L)]
            for e in range(L):
                ucopies.append(pltpu.make_async_copy(
                    utab_hbm.at[vu[e], :], ubuf_v.at[:, k * L + e], usem))
        for c in ucopies:
            c.start()
        icopies = []
        for k in range(W // L):
            vi = iidx_v[w, pl.ds(k * L, L)]
            for e in range(L):
                icopies.append(pltpu.make_async_copy(
                    itab_hbm.at[vi[e], :], ibuf_v.at[:, k * L + e], isem))
        for c in icopies:
            c.start()
        for c in ucopies:
            c.wait()
        pltpu.make_async_copy(
            ubuf_v.at[0], uout_hbm.at[pl.ds(pos, W)], osem).start()
        for c in icopies:
            c.wait()
        pltpu.make_async_copy(
            ibuf_v.at[0], iout_hbm.at[pl.ds(pos, W)], osem).start()
        pltpu.make_async_copy(
            ubuf_v.at[0], uout_hbm.at[pl.ds(pos, W)], osem).wait()
        pltpu.make_async_copy(
            ibuf_v.at[0], iout_hbm.at[pl.ds(pos, W)], osem).wait()
        return carry

    lax.fori_loop(0, NWIN, window, 0)


def _lookup_sc(uidx, iidx, utab, itab, uv, iv):
    mesh = plsc.VectorSubcoreMesh(core_axis_name="c", subcore_axis_name="s")
    f = pl.kernel(
        functools.partial(_gather_body, uv, iv),
        mesh=mesh,
        out_type=(
            jax.ShapeDtypeStruct((BATCH,), jnp.float32),
            jax.ShapeDtypeStruct((BATCH,), jnp.float32),
        ),
        scratch_types=[
            pltpu.VMEM((NWIN, W), jnp.int32),
            pltpu.VMEM((NWIN, W), jnp.int32),
            pltpu.VMEM((W, 64), jnp.float32),
            pltpu.VMEM((W, 64), jnp.float32),
            pltpu.SemaphoreType.DMA,
            pltpu.SemaphoreType.DMA,
            pltpu.SemaphoreType.DMA,
        ],
    )
    return f(uidx, iidx, utab, itab)


def kernel(x, uid_weight, iid_weight):
    uidx = x[:, 0].reshape(BATCH // W, W)
    iidx = x[:, 1].reshape(BATCH // W, W)
    uvals, ivals = _lookup_sc(uidx, iidx, uid_weight, iid_weight,
                              uid_weight.shape[0], iid_weight.shape[0])
    return jnp.stack([uvals, ivals], axis=1)


# speedup vs baseline: 10.2873x; 8.7448x over previous
import functools

import jax
import jax.numpy as jnp
from jax import lax
from jax.experimental import pallas as pl
from jax.experimental.pallas import tpu as pltpu
from jax.experimental.pallas import tpu_sc as plsc

NC = 2
NS = 16
NW = NC * NS
L = 16

BATCH = 16384
B_PER_W = BATCH // NW
ROWS_PER_W = B_PER_W // 128


def _gather_body(uv, iv, uidx_hbm, iidx_hbm, ucol_hbm, icol_hbm,
                 uout_hbm, iout_hbm, uidx_v, iidx_v, uval_v, ival_v, sem):
    wid = lax.axis_index("s") * NC + lax.axis_index("c")
    row0 = wid * ROWS_PER_W

    pltpu.sync_copy(uidx_hbm.at[pl.ds(row0, ROWS_PER_W), :], uidx_v)
    pltpu.sync_copy(iidx_hbm.at[pl.ds(row0, ROWS_PER_W), :], iidx_v)

    for j in range(ROWS_PER_W):
        for k in range(128 // L):
            u = uidx_v[j, pl.ds(k * L, L)]
            uidx_v[j, pl.ds(k * L, L)] = jnp.clip(u, 0, uv - 1)
            i = iidx_v[j, pl.ds(k * L, L)]
            iidx_v[j, pl.ds(k * L, L)] = jnp.clip(i, 0, iv - 1)

    copies = []
    for j in range(ROWS_PER_W):
        copies.append(pltpu.make_async_copy(
            ucol_hbm.at[uidx_v.at[j]], uval_v.at[j], sem))
        copies.append(pltpu.make_async_copy(
            icol_hbm.at[iidx_v.at[j]], ival_v.at[j], sem))
    for c in copies:
        c.start()
    for c in copies:
        c.wait()

    pltpu.sync_copy(uval_v, uout_hbm.at[pl.ds(row0, ROWS_PER_W), :])
    pltpu.sync_copy(ival_v, iout_hbm.at[pl.ds(row0, ROWS_PER_W), :])


def _lookup_sc(uidx, iidx, ucol, icol, uv, iv):
    mesh = plsc.VectorSubcoreMesh(core_axis_name="c", subcore_axis_name="s")
    f = pl.kernel(
        functools.partial(_gather_body, uv, iv),
        mesh=mesh,
        out_type=(
            jax.ShapeDtypeStruct((BATCH // 128, 128), jnp.float32),
            jax.ShapeDtypeStruct((BATCH // 128, 128), jnp.float32),
        ),
        scratch_types=[
            pltpu.VMEM((ROWS_PER_W, 128), jnp.int32),
            pltpu.VMEM((ROWS_PER_W, 128), jnp.int32),
            pltpu.VMEM((ROWS_PER_W, 128), jnp.float32),
            pltpu.VMEM((ROWS_PER_W, 128), jnp.float32),
            pltpu.SemaphoreType.DMA,
        ],
    )
    return f(uidx, iidx, ucol, icol)


def kernel(x, uid_weight, iid_weight):
    uidx = x[:, 0].reshape(BATCH // 128, 128)
    iidx = x[:, 1].reshape(BATCH // 128, 128)
    ucol = uid_weight[:, 0]
    icol = iid_weight[:, 0]
    uvals, ivals = _lookup_sc(uidx, iidx, ucol, icol,
                              uid_weight.shape[0], iid_weight.shape[0])
    return jnp.stack([uvals.reshape(BATCH), ivals.reshape(BATCH)], axis=1)
